# baseline (speedup 1.0000x reference)
import jax
import jax.numpy as jnp
import numpy as np
from jax import lax
from jax.experimental import pallas as pl
from jax.experimental.pallas import tpu as pltpu
from jax.experimental.pallas import tpu_sc as plsc

_N = 100
_G = 7
_W = 128

_SEL_CACHE = {}


def _selected_batches(batch):
    if batch not in _SEL_CACHE:
        with jax.ensure_compile_time_eval():
            key = jax.random.key(42)
            _SEL_CACHE[batch] = np.asarray(
                jnp.sort(jax.random.randint(key, (_N,), 0, batch)))
    return _SEL_CACHE[batch]


def _sc_body(xp_ref, sel_ref, cm_ref, nd_out, box_out, sc_out, cls_out,
             xin_v, sel_v, cm_v, s_v, box_v, sc_v, cls_v, nd_v, sem):
    nbatch = xp_ref.shape[0]
    b = lax.axis_index("s") * 2 + lax.axis_index("c")

    @pl.when(b < nbatch)
    def _():
        d_in = [pltpu.async_copy(xp_ref.at[b], xin_v, sem),
                pltpu.async_copy(sel_ref, sel_v, sem),
                pltpu.async_copy(cm_ref, cm_v, sem)]
        for d in d_in:
            d.wait()

        iota = lax.iota(jnp.int32, 16)
        bvec = jnp.full((16,), 0, jnp.int32) + b

        cmc = [[cm_v[4 * k + c, :] for c in range(4)] for k in range(4)]

        e_idx, s_list, box_list, lab_list, valid_list = [], [], [], [], []
        for j in range(_G):
            e_j = iota + 16 * j
            sel_j = sel_v[pl.ds(16 * j, 16)]
            live = sel_j == bvec
            X = [plsc.load_gather(xin_v, [e_j * 8 + ch]) for ch in range(6)]
            prod = X[4] * X[5]
            fill = jnp.where(e_j <= _N, jnp.full((16,), 0.0, jnp.float32),
                             jnp.full((16,), -1.0, jnp.float32))
            s_j = jnp.where(live, prod, fill)
            s_v[pl.ds(16 * j, 16)] = s_j
            zf = jnp.full((16,), 0.0, jnp.float32)
            box_j = [jnp.where(live,
                               X[0] * cmc[0][c] + X[1] * cmc[1][c]
                               + X[2] * cmc[2][c] + X[3] * cmc[3][c], zf)
                     for c in range(4)]
            lab_j = jnp.where(live, jnp.full((16,), 0, jnp.int32),
                              jnp.full((16,), -1, jnp.int32))
            e_idx.append(e_j)
            s_list.append(s_j)
            box_list.append(box_j)
            lab_list.append(lab_j)
            valid_list.append(e_j <= _N)

        def rank_step(k, ranks):
            kv = jnp.full((16,), 0, jnp.int32) + k
            skv = plsc.load_gather(s_v, [kv])
            out = []
            for j in range(_G):
                before = (skv > s_list[j]) | ((skv == s_list[j])
                                              & (kv < e_idx[j]))
                out.append(ranks[j] + before.astype(jnp.int32))
            return tuple(out)

        ranks = lax.fori_loop(
            0, _N + 1, rank_step,
            tuple(jnp.full((16,), 0, jnp.int32) for _ in range(_G)),
            unroll=4)

        cnt = jnp.full((16,), 0, jnp.int32)
        for j in range(_G):
            plsc.store_scatter(sc_v, [ranks[j]], s_list[j],
                               mask=valid_list[j])
            plsc.store_scatter(cls_v, [ranks[j]], lab_list[j],
                               mask=valid_list[j])
            for c in range(4):
                plsc.store_scatter(
                    box_v, [ranks[j], jnp.full((16,), c, jnp.int32)],
                    box_list[j][c], mask=valid_list[j])
            cnt = cnt + (s_list[j] > 0).astype(jnp.int32)

        nd_v[...] = jnp.full((16,), 0, jnp.int32) + jnp.sum(cnt)

        d_out = [pltpu.async_copy(nd_v, nd_out.at[b], sem),
                 pltpu.async_copy(box_v, box_out.at[b], sem),
                 pltpu.async_copy(sc_v, sc_out.at[b], sem),
                 pltpu.async_copy(cls_v, cls_out.at[b], sem)]
        for d in d_out:
            d.wait()


def kernel(x, convert_matrix):
    batch = x.shape[0]
    f32, i32 = jnp.float32, jnp.int32

    sel_np = np.full((_W,), batch + 7, np.int32)
    sel_np[:_N] = _selected_batches(batch)
    sel_pad = jnp.asarray(sel_np)

    xs = lax.slice(x, (0, _N, 0), (batch, 2 * _N, 8)).reshape(batch, 8 * _N)
    xp = jnp.zeros((batch, 1024), f32).at[:, :8 * _N].set(xs)
    cm_bc = jnp.broadcast_to(
        convert_matrix.astype(f32).reshape(16)[:, None], (16, 16))

    out_type = (
        jax.ShapeDtypeStruct((batch, 16), i32),
        jax.ShapeDtypeStruct((batch, _W, 4), f32),
        jax.ShapeDtypeStruct((batch, _W), f32),
        jax.ShapeDtypeStruct((batch, _W), i32),
    )
    scratch_types = [
        pltpu.VMEM((1024,), f32),
        pltpu.VMEM((_W,), i32),
        pltpu.VMEM((16, 16), f32),
        pltpu.VMEM((_W,), f32),
        pltpu.VMEM((_W, 4), f32),
        pltpu.VMEM((_W,), f32),
        pltpu.VMEM((_W,), i32),
        pltpu.VMEM((16,), i32),
        pltpu.SemaphoreType.DMA,
    ]
    mesh = plsc.VectorSubcoreMesh(core_axis_name="c", subcore_axis_name="s")
    nd, boxp, scp, clsp = pl.kernel(
        _sc_body, out_type=out_type, scratch_types=scratch_types,
        mesh=mesh,
        compiler_params=pltpu.CompilerParams(needs_layout_passes=False),
    )(xp, sel_pad, cm_bc)

    return (nd[:, :1], boxp[:, :_N + 1, :], scp[:, :_N + 1],
            clsp[:, :_N + 1])


# speedup vs baseline: 1.2708x; 1.2708x over previous
import jax
import jax.numpy as jnp
import numpy as np
from jax import lax
from jax.experimental import pallas as pl
from jax.experimental.pallas import tpu as pltpu
from jax.experimental.pallas import tpu_sc as plsc

_N = 100
_G = 7
_W = 128

_SEL_CACHE = {}


def _selected_batches(batch):
    if batch not in _SEL_CACHE:
        with jax.ensure_compile_time_eval():
            key = jax.random.key(42)
            _SEL_CACHE[batch] = np.asarray(
                jnp.sort(jax.random.randint(key, (_N,), 0, batch)))
    return _SEL_CACHE[batch]


def _sc_body(xp_ref, sel_ref, cm_ref, nd_out, box_out, sc_out, cls_out,
             xin_v, sel_v, cm_v, s_v, box_v, sc_v, cls_v, nd_v, sem):
    nbatch = xp_ref.shape[0]
    b = lax.axis_index("s") * 2 + lax.axis_index("c")

    @pl.when(b < nbatch)
    def _():
        d_in = [pltpu.async_copy(xp_ref.at[b], xin_v, sem),
                pltpu.async_copy(sel_ref, sel_v, sem),
                pltpu.async_copy(cm_ref, cm_v, sem)]
        for d in d_in:
            d.wait()

        iota = lax.iota(jnp.int32, 16)
        bvec = jnp.full((16,), 0, jnp.int32) + b

        cmc = [[cm_v[4 * k + c, :] for c in range(4)] for k in range(4)]

        e_idx, s_list, box_list, lab_list, valid_list = [], [], [], [], []
        for j in range(_G):
            e_j = iota + 16 * j
            sel_j = sel_v[pl.ds(16 * j, 16)]
            live = sel_j == bvec
            X = [plsc.load_gather(xin_v, [e_j * 8 + ch]) for ch in range(6)]
            prod = X[4] * X[5]
            fill = jnp.where(e_j <= _N, jnp.full((16,), 0.0, jnp.float32),
                             jnp.full((16,), -1.0, jnp.float32))
            s_j = jnp.where(live, prod, fill)
            s_v[pl.ds(16 * j, 16)] = s_j
            zf = jnp.full((16,), 0.0, jnp.float32)
            box_j = [jnp.where(live,
                               X[0] * cmc[0][c] + X[1] * cmc[1][c]
                               + X[2] * cmc[2][c] + X[3] * cmc[3][c], zf)
                     for c in range(4)]
            lab_j = jnp.where(live, jnp.full((16,), 0, jnp.int32),
                              jnp.full((16,), -1, jnp.int32))
            e_idx.append(e_j)
            s_list.append(s_j)
            box_list.append(box_j)
            lab_list.append(lab_j)
            valid_list.append(e_j <= _N)

        def rank_step(k, ranks):
            kv = jnp.full((16,), 0, jnp.int32) + k
            skv = plsc.load_gather(s_v, [kv])
            out = []
            for j in range(_G):
                before = (skv > s_list[j]) | ((skv == s_list[j])
                                              & (kv < e_idx[j]))
                out.append(ranks[j] + before.astype(jnp.int32))
            return tuple(out)

        ranks = lax.fori_loop(
            0, _N + 1, rank_step,
            tuple(jnp.full((16,), 0, jnp.int32) for _ in range(_G)))

        cnt = jnp.full((16,), 0, jnp.int32)
        for j in range(_G):
            plsc.store_scatter(sc_v, [ranks[j]], s_list[j],
                               mask=valid_list[j])
            plsc.store_scatter(cls_v, [ranks[j]], lab_list[j],
                               mask=valid_list[j])
            for c in range(4):
                plsc.store_scatter(
                    box_v, [ranks[j], jnp.full((16,), c, jnp.int32)],
                    box_list[j][c], mask=valid_list[j])
            cnt = cnt + (s_list[j] > 0).astype(jnp.int32)

        nd_v[...] = jnp.full((16,), 0, jnp.int32) + jnp.sum(cnt)

        d_out = [pltpu.async_copy(nd_v, nd_out.at[b], sem),
                 pltpu.async_copy(box_v, box_out.at[b], sem),
                 pltpu.async_copy(sc_v, sc_out.at[b], sem),
                 pltpu.async_copy(cls_v, cls_out.at[b], sem)]
        for d in d_out:
            d.wait()


def kernel(x, convert_matrix):
    batch = x.shape[0]
    f32, i32 = jnp.float32, jnp.int32

    sel_np = np.full((_W,), batch + 7, np.int32)
    sel_np[:_N] = _selected_batches(batch)
    sel_pad = jnp.asarray(sel_np)

    xs = lax.slice(x, (0, _N, 0), (batch, 2 * _N, 8)).reshape(batch, 8 * _N)
    xp = jnp.zeros((batch, 1024), f32).at[:, :8 * _N].set(xs)
    cm_bc = jnp.broadcast_to(
        convert_matrix.astype(f32).reshape(16)[:, None], (16, 16))

    out_type = (
        jax.ShapeDtypeStruct((batch, 16), i32),
        jax.ShapeDtypeStruct((batch, _W, 4), f32),
        jax.ShapeDtypeStruct((batch, _W), f32),
        jax.ShapeDtypeStruct((batch, _W), i32),
    )
    scratch_types = [
        pltpu.VMEM((1024,), f32),
        pltpu.VMEM((_W,), i32),
        pltpu.VMEM((16, 16), f32),
        pltpu.VMEM((_W,), f32),
        pltpu.VMEM((_W, 4), f32),
        pltpu.VMEM((_W,), f32),
        pltpu.VMEM((_W,), i32),
        pltpu.VMEM((16,), i32),
        pltpu.SemaphoreType.DMA,
    ]
    mesh = plsc.VectorSubcoreMesh(core_axis_name="c", subcore_axis_name="s")
    nd, boxp, scp, clsp = pl.kernel(
        _sc_body, out_type=out_type, scratch_types=scratch_types,
        mesh=mesh,
        compiler_params=pltpu.CompilerParams(needs_layout_passes=False),
    )(xp, sel_pad, cm_bc)

    return (nd[:, :1], boxp[:, :_N + 1, :], scp[:, :_N + 1],
            clsp[:, :_N + 1])


# speedup vs baseline: 1.3479x; 1.0606x over previous
import jax
import jax.numpy as jnp
import numpy as np
from jax import lax
from jax.experimental import pallas as pl
from jax.experimental.pallas import tpu as pltpu
from jax.experimental.pallas import tpu_sc as plsc

_N = 100
_G = 7
_W = 128

_SEL_CACHE = {}


def _selected_batches(batch):
    if batch not in _SEL_CACHE:
        with jax.ensure_compile_time_eval():
            key = jax.random.key(42)
            _SEL_CACHE[batch] = np.asarray(
                jnp.sort(jax.random.randint(key, (_N,), 0, batch)))
    return _SEL_CACHE[batch]


def _sc_body(xp_ref, sel_ref, cm_ref, nd_out, box_out, sc_out, cls_out,
             xin_v, sel_v, cm_v, s_v, box_v, sc_v, cls_v, nd_v, sem):
    nbatch = xp_ref.shape[0]
    b = lax.axis_index("s")

    @pl.when(b < nbatch)
    def _():
        d_in = [pltpu.async_copy(xp_ref.at[b], xin_v, sem),
                pltpu.async_copy(sel_ref, sel_v, sem),
                pltpu.async_copy(cm_ref, cm_v, sem)]
        for d in d_in:
            d.wait()

        iota = lax.iota(jnp.int32, 16)
        bvec = jnp.full((16,), 0, jnp.int32) + b

        cmc = [[cm_v[4 * k + c, :] for c in range(4)] for k in range(4)]

        e_idx, s_list, box_list, lab_list, valid_list = [], [], [], [], []
        for j in range(_G):
            e_j = iota + 16 * j
            sel_j = sel_v[pl.ds(16 * j, 16)]
            live = sel_j == bvec
            X = [plsc.load_gather(xin_v, [e_j * 8 + ch]) for ch in range(6)]
            prod = X[4] * X[5]
            fill = jnp.where(e_j <= _N, jnp.full((16,), 0.0, jnp.float32),
                             jnp.full((16,), -1.0, jnp.float32))
            s_j = jnp.where(live, prod, fill)
            s_v[pl.ds(16 * j, 16)] = s_j
            zf = jnp.full((16,), 0.0, jnp.float32)
            box_j = [jnp.where(live,
                               X[0] * cmc[0][c] + X[1] * cmc[1][c]
                               + X[2] * cmc[2][c] + X[3] * cmc[3][c], zf)
                     for c in range(4)]
            lab_j = jnp.where(live, jnp.full((16,), 0, jnp.int32),
                              jnp.full((16,), -1, jnp.int32))
            e_idx.append(e_j)
            s_list.append(s_j)
            box_list.append(box_j)
            lab_list.append(lab_j)
            valid_list.append(e_j <= _N)

        def rank_step(k, ranks):
            kv = jnp.full((16,), 0, jnp.int32) + k
            skv = plsc.load_gather(s_v, [kv])
            out = []
            for j in range(_G):
                before = (skv > s_list[j]) | ((skv == s_list[j])
                                              & (kv < e_idx[j]))
                out.append(ranks[j] + before.astype(jnp.int32))
            return tuple(out)

        ranks = lax.fori_loop(
            0, _N + 1, rank_step,
            tuple(jnp.full((16,), 0, jnp.int32) for _ in range(_G)))

        cnt = jnp.full((16,), 0, jnp.int32)
        for j in range(_G):
            plsc.store_scatter(sc_v, [ranks[j]], s_list[j],
                               mask=valid_list[j])
            plsc.store_scatter(cls_v, [ranks[j]], lab_list[j],
                               mask=valid_list[j])
            for c in range(4):
                plsc.store_scatter(
                    box_v, [ranks[j], jnp.full((16,), c, jnp.int32)],
                    box_list[j][c], mask=valid_list[j])
            cnt = cnt + (s_list[j] > 0).astype(jnp.int32)

        nd_v[...] = jnp.full((16,), 0, jnp.int32) + jnp.sum(cnt)

        d_out = [pltpu.async_copy(nd_v, nd_out.at[b], sem),
                 pltpu.async_copy(box_v, box_out.at[b], sem),
                 pltpu.async_copy(sc_v, sc_out.at[b], sem),
                 pltpu.async_copy(cls_v, cls_out.at[b], sem)]
        for d in d_out:
            d.wait()


def kernel(x, convert_matrix):
    batch = x.shape[0]
    f32, i32 = jnp.float32, jnp.int32

    sel_np = np.full((_W,), batch + 7, np.int32)
    sel_np[:_N] = _selected_batches(batch)
    sel_pad = jnp.asarray(sel_np)

    xs = lax.slice(x, (0, _N, 0), (batch, 2 * _N, 8)).reshape(batch, 8 * _N)
    xp = jnp.zeros((batch, 1024), f32).at[:, :8 * _N].set(xs)
    cm_bc = jnp.broadcast_to(
        convert_matrix.astype(f32).reshape(16)[:, None], (16, 16))

    out_type = (
        jax.ShapeDtypeStruct((batch, 16), i32),
        jax.ShapeDtypeStruct((batch, _W, 4), f32),
        jax.ShapeDtypeStruct((batch, _W), f32),
        jax.ShapeDtypeStruct((batch, _W), i32),
    )
    scratch_types = [
        pltpu.VMEM((1024,), f32),
        pltpu.VMEM((_W,), i32),
        pltpu.VMEM((16, 16), f32),
        pltpu.VMEM((_W,), f32),
        pltpu.VMEM((_W, 4), f32),
        pltpu.VMEM((_W,), f32),
        pltpu.VMEM((_W,), i32),
        pltpu.VMEM((16,), i32),
        pltpu.SemaphoreType.DMA,
    ]
    mesh = plsc.VectorSubcoreMesh(core_axis_name="c", subcore_axis_name="s",
                                  num_cores=1)
    nd, boxp, scp, clsp = pl.kernel(
        _sc_body, out_type=out_type, scratch_types=scratch_types,
        mesh=mesh,
        compiler_params=pltpu.CompilerParams(needs_layout_passes=False),
    )(xp, sel_pad, cm_bc)

    return (nd[:, :1], boxp[:, :_N + 1, :], scp[:, :_N + 1],
            clsp[:, :_N + 1])


# speedup vs baseline: 1.4818x; 1.0994x over previous
import jax
import jax.numpy as jnp
import numpy as np
from jax import lax
from jax.experimental import pallas as pl
from jax.experimental.pallas import tpu as pltpu
from jax.experimental.pallas import tpu_sc as plsc

_N = 100
_G = 7
_W = 128
_CM0 = 896
_SEL0 = 1152
_XPW = 1280

_SEL_CACHE = {}


def _selected_batches(batch):
    if batch not in _SEL_CACHE:
        with jax.ensure_compile_time_eval():
            key = jax.random.key(42)
            _SEL_CACHE[batch] = np.asarray(
                jnp.sort(jax.random.randint(key, (_N,), 0, batch)))
    return _SEL_CACHE[batch]


def _sc_body(xp_ref, nd_out, box_out, sc_out, cls_out,
             xin_v, s_v, box_v, sc_v, cls_v, nd_v, sem):
    nbatch = xp_ref.shape[0]
    b = lax.axis_index("s")

    @pl.when(b < nbatch)
    def _():
        pltpu.async_copy(xp_ref.at[b], xin_v, sem).wait()

        iota = lax.iota(jnp.int32, 16)
        bvec = jnp.full((16,), 0, jnp.int32) + b

        cmc = [[xin_v[pl.ds(_CM0 + 16 * (4 * k + c), 16)] for c in range(4)]
               for k in range(4)]

        e_idx, s_list, box_list, lab_list, valid_list = [], [], [], [], []
        for j in range(_G):
            e_j = iota + 16 * j
            sel_j = plsc.bitcast(xin_v[pl.ds(_SEL0 + 16 * j, 16)], jnp.int32)
            live = sel_j == bvec
            X = [plsc.load_gather(xin_v, [e_j * 8 + ch]) for ch in range(6)]
            prod = X[4] * X[5]
            fill = jnp.where(e_j <= _N, jnp.full((16,), 0.0, jnp.float32),
                             jnp.full((16,), -1.0, jnp.float32))
            s_j = jnp.where(live, prod, fill)
            s_v[pl.ds(16 * j, 16)] = s_j
            zf = jnp.full((16,), 0.0, jnp.float32)
            box_j = [jnp.where(live,
                               X[0] * cmc[0][c] + X[1] * cmc[1][c]
                               + X[2] * cmc[2][c] + X[3] * cmc[3][c], zf)
                     for c in range(4)]
            lab_j = jnp.where(live, jnp.full((16,), 0, jnp.int32),
                              jnp.full((16,), -1, jnp.int32))
            e_idx.append(e_j)
            s_list.append(s_j)
            box_list.append(box_j)
            lab_list.append(lab_j)
            valid_list.append(e_j <= _N)

        def rank_step(k, ranks):
            kv = jnp.full((16,), 0, jnp.int32) + k
            skv = plsc.load_gather(s_v, [kv])
            out = []
            for j in range(_G):
                before = (skv > s_list[j]) | ((skv == s_list[j])
                                              & (kv < e_idx[j]))
                out.append(ranks[j] + before.astype(jnp.int32))
            return tuple(out)

        ranks = lax.fori_loop(
            0, _N + 1, rank_step,
            tuple(jnp.full((16,), 0, jnp.int32) for _ in range(_G)))

        cnt = jnp.full((16,), 0, jnp.int32)
        for j in range(_G):
            plsc.store_scatter(sc_v, [ranks[j]], s_list[j],
                               mask=valid_list[j])
            plsc.store_scatter(cls_v, [ranks[j]], lab_list[j],
                               mask=valid_list[j])
            for c in range(4):
                plsc.store_scatter(
                    box_v, [ranks[j], jnp.full((16,), c, jnp.int32)],
                    box_list[j][c], mask=valid_list[j])
            cnt = cnt + (s_list[j] > 0).astype(jnp.int32)

        nd_v[...] = jnp.full((16,), 0, jnp.int32) + jnp.sum(cnt)

        d_out = [pltpu.async_copy(nd_v, nd_out.at[b], sem),
                 pltpu.async_copy(box_v, box_out.at[b], sem),
                 pltpu.async_copy(sc_v, sc_out.at[b], sem),
                 pltpu.async_copy(cls_v, cls_out.at[b], sem)]
        for d in d_out:
            d.wait()


def kernel(x, convert_matrix):
    batch = x.shape[0]
    f32, i32 = jnp.float32, jnp.int32

    sel_np = np.full((_W,), batch + 7, np.int32)
    sel_np[:_N] = _selected_batches(batch)
    sel_bits = jnp.asarray(sel_np.view(np.float32))

    xs = lax.slice(x, (0, _N, 0), (batch, 2 * _N, 8)).reshape(batch, 8 * _N)
    cm_bc = jnp.broadcast_to(
        convert_matrix.astype(f32).reshape(16)[:, None], (16, 16)
    ).reshape(256)
    tail = jnp.concatenate(
        [jnp.zeros((_CM0 - 8 * _N,), f32), cm_bc, sel_bits])
    xp = jnp.concatenate(
        [xs, jnp.broadcast_to(tail[None], (batch, _XPW - 8 * _N))], axis=1)

    out_type = (
        jax.ShapeDtypeStruct((batch, 16), i32),
        jax.ShapeDtypeStruct((batch, _W, 4), f32),
        jax.ShapeDtypeStruct((batch, _W), f32),
        jax.ShapeDtypeStruct((batch, _W), i32),
    )
    scratch_types = [
        pltpu.VMEM((_XPW,), f32),
        pltpu.VMEM((_W,), f32),
        pltpu.VMEM((_W, 4), f32),
        pltpu.VMEM((_W,), f32),
        pltpu.VMEM((_W,), i32),
        pltpu.VMEM((16,), i32),
        pltpu.SemaphoreType.DMA,
    ]
    mesh = plsc.VectorSubcoreMesh(core_axis_name="c", subcore_axis_name="s",
                                  num_cores=1)
    nd, boxp, scp, clsp = pl.kernel(
        _sc_body, out_type=out_type, scratch_types=scratch_types,
        mesh=mesh,
        compiler_params=pltpu.CompilerParams(needs_layout_passes=False),
    )(xp)

    return (nd[:, :1], boxp[:, :_N + 1, :], scp[:, :_N + 1],
            clsp[:, :_N + 1])


# speedup vs baseline: 1.4865x; 1.0032x over previous
import jax
import jax.numpy as jnp
import numpy as np
from jax import lax
from jax.experimental import pallas as pl
from jax.experimental.pallas import tpu as pltpu
from jax.experimental.pallas import tpu_sc as plsc

_N = 100
_G = 7
_W = 128
_CM0 = 896
_SEL0 = 1152
_XPW = 1280

_SEL_CACHE = {}


def _selected_batches(batch):
    if batch not in _SEL_CACHE:
        with jax.ensure_compile_time_eval():
            key = jax.random.key(42)
            _SEL_CACHE[batch] = np.asarray(
                jnp.sort(jax.random.randint(key, (_N,), 0, batch)))
    return _SEL_CACHE[batch]


def _sc_body(xp_ref, nd_out, box_out, sc_out, cls_out,
             xin_v, s_v, box_v, sc_v, cls_v, nd_v, sem):
    nbatch = xp_ref.shape[0]
    b = lax.axis_index("s")

    @pl.when(b < nbatch)
    def _():
        pltpu.async_copy(xp_ref.at[b], xin_v, sem).wait()

        iota = lax.iota(jnp.int32, 16)
        bvec = jnp.full((16,), 0, jnp.int32) + b

        cmc = [[xin_v[pl.ds(_CM0 + 16 * (4 * k + c), 16)] for c in range(4)]
               for k in range(4)]

        e_idx, s_list, box_list, lab_list, valid_list = [], [], [], [], []
        for j in range(_G):
            e_j = iota + 16 * j
            sel_j = xin_v[pl.ds(_SEL0 + 16 * j, 16)].astype(jnp.int32)
            live = sel_j == bvec
            X = [plsc.load_gather(xin_v, [e_j * 8 + ch]) for ch in range(6)]
            prod = X[4] * X[5]
            fill = jnp.where(e_j <= _N, jnp.full((16,), 0.0, jnp.float32),
                             jnp.full((16,), -1.0, jnp.float32))
            s_j = jnp.where(live, prod, fill)
            s_v[pl.ds(16 * j, 16)] = s_j
            zf = jnp.full((16,), 0.0, jnp.float32)
            box_j = [jnp.where(live,
                               X[0] * cmc[0][c] + X[1] * cmc[1][c]
                               + X[2] * cmc[2][c] + X[3] * cmc[3][c], zf)
                     for c in range(4)]
            lab_j = jnp.where(live, jnp.full((16,), 0, jnp.int32),
                              jnp.full((16,), -1, jnp.int32))
            e_idx.append(e_j)
            s_list.append(s_j)
            box_list.append(box_j)
            lab_list.append(lab_j)
            valid_list.append(e_j <= _N)

        def rank_step(k, ranks):
            kv = jnp.full((16,), 0, jnp.int32) + k
            skv = plsc.load_gather(s_v, [kv])
            out = []
            for j in range(_G):
                before = (skv > s_list[j]) | ((skv == s_list[j])
                                              & (kv < e_idx[j]))
                out.append(ranks[j] + before.astype(jnp.int32))
            return tuple(out)

        ranks = lax.fori_loop(
            0, _N + 1, rank_step,
            tuple(jnp.full((16,), 0, jnp.int32) for _ in range(_G)))

        cnt = jnp.full((16,), 0, jnp.int32)
        for j in range(_G):
            plsc.store_scatter(sc_v, [ranks[j]], s_list[j],
                               mask=valid_list[j])
            plsc.store_scatter(cls_v, [ranks[j]], lab_list[j],
                               mask=valid_list[j])
            for c in range(4):
                plsc.store_scatter(
                    box_v, [ranks[j], jnp.full((16,), c, jnp.int32)],
                    box_list[j][c], mask=valid_list[j])
            cnt = cnt + (s_list[j] > 0).astype(jnp.int32)

        nd_v[...] = jnp.full((16,), 0, jnp.int32) + jnp.sum(cnt)

        d_out = [pltpu.async_copy(nd_v, nd_out.at[b], sem),
                 pltpu.async_copy(box_v, box_out.at[b], sem),
                 pltpu.async_copy(sc_v, sc_out.at[b], sem),
                 pltpu.async_copy(cls_v, cls_out.at[b], sem)]
        for d in d_out:
            d.wait()


def kernel(x, convert_matrix):
    batch = x.shape[0]
    f32, i32 = jnp.float32, jnp.int32

    sel_np = np.full((_W,), batch + 7, np.int32)
    sel_np[:_N] = _selected_batches(batch)
    sel_f = jnp.asarray(sel_np.astype(np.float32))

    xs = lax.slice(x, (0, _N, 0), (batch, 2 * _N, 8)).reshape(batch, 8 * _N)
    cm_bc = jnp.broadcast_to(
        convert_matrix.astype(f32).reshape(16)[:, None], (16, 16)
    ).reshape(256)
    tail = jnp.concatenate(
        [jnp.zeros((_CM0 - 8 * _N,), f32), cm_bc, sel_f])
    xp = jnp.concatenate(
        [xs, jnp.broadcast_to(tail[None], (batch, _XPW - 8 * _N))], axis=1)

    out_type = (
        jax.ShapeDtypeStruct((batch, 16), i32),
        jax.ShapeDtypeStruct((batch, _W, 4), f32),
        jax.ShapeDtypeStruct((batch, _W), f32),
        jax.ShapeDtypeStruct((batch, _W), i32),
    )
    scratch_types = [
        pltpu.VMEM((_XPW,), f32),
        pltpu.VMEM((_W,), f32),
        pltpu.VMEM((_W, 4), f32),
        pltpu.VMEM((_W,), f32),
        pltpu.VMEM((_W,), i32),
        pltpu.VMEM((16,), i32),
        pltpu.SemaphoreType.DMA,
    ]
    mesh = plsc.VectorSubcoreMesh(core_axis_name="c", subcore_axis_name="s",
                                  num_cores=1)
    nd, boxp, scp, clsp = pl.kernel(
        _sc_body, out_type=out_type, scratch_types=scratch_types,
        mesh=mesh,
        compiler_params=pltpu.CompilerParams(needs_layout_passes=False),
    )(xp)

    return (nd[:, :1], boxp[:, :_N + 1, :], scp[:, :_N + 1],
            clsp[:, :_N + 1])
